# baseline (speedup 1.0000x reference)
import functools

import jax
import jax.numpy as jnp
from jax import lax
from jax.experimental import pallas as pl
from jax.experimental.pallas import tpu as pltpu
from jax.experimental.pallas import tpu_sc as plsc

_N_CODES = 8192
_D = 256
_BT = 1024
_BC = 256


def _dist_argmin_body(a_ref, x_ref, t_ref, c_ref, ci_ref, idx_ref):
    xb = x_ref[...]
    a = a_ref[...]
    n_chunks = _N_CODES // _BC

    def dist_at(base):
        tb = t_ref[pl.ds(base, _BC), :]
        prod = lax.dot_general(
            xb, tb, (((1,), (1,)), ((), ())),
            preferred_element_type=jnp.float32,
        )
        return (a - prod) + c_ref[:, pl.ds(base, _BC)]

    def merged_pair(base):
        d0 = dist_at(base)
        d1 = dist_at(base + _BC)
        i0 = jnp.broadcast_to(ci_ref[:, pl.ds(base, _BC)], (_BT, _BC))
        i1 = jnp.broadcast_to(ci_ref[:, pl.ds(base + _BC, _BC)],
                              (_BT, _BC))
        m = d1 < d0
        return jnp.where(m, d1, d0), jnp.where(m, i1, i0)

    def merged_quad(base):
        d0, i0 = merged_pair(base)
        d1, i1 = merged_pair(base + 2 * _BC)
        m = d1 < d0
        return jnp.where(m, d1, d0), jnp.where(m, i1, i0)

    def merged_oct(base):
        d0, i0 = merged_quad(base)
        d1, i1 = merged_quad(base + 4 * _BC)
        m = d1 < d0
        return jnp.where(m, d1, d0), jnp.where(m, i1, i0)

    def merge(a0, a1):
        (d0, i0), (d1, i1) = a0, a1
        m = d1 < d0
        return jnp.where(m, d1, d0), jnp.where(m, i1, i0)

    level = [merged_oct(k * 8 * _BC) for k in range(n_chunks // 8)]
    while len(level) > 1:
        level = [merge(level[k], level[k + 1])
                 for k in range(0, len(level), 2)]
    best, bidx = level[0]
    gmin = jnp.min(best, axis=1, keepdims=True)
    cand = jnp.where(best == gmin, bidx, jnp.float32(_N_CODES))
    idx_ref[...] = jnp.min(cand, axis=1).reshape(_BT, 1).astype(jnp.int32)


def _argmin_codes(x2d, templat, a, c):
    n_tok = x2d.shape[0]
    xb16 = (2.0 * x2d).astype(jnp.bfloat16)
    tb16 = templat.astype(jnp.bfloat16)
    ci = lax.iota(jnp.float32, _N_CODES)[None, :]
    grid = (n_tok // _BT,)
    return pl.pallas_call(
        _dist_argmin_body,
        grid=grid,
        in_specs=[
            pl.BlockSpec((_BT, 1), lambda i: (i, 0)),
            pl.BlockSpec((_BT, _D), lambda i: (i, 0)),
            pl.BlockSpec((_N_CODES, _D), lambda i: (0, 0)),
            pl.BlockSpec((1, _N_CODES), lambda i: (0, 0)),
            pl.BlockSpec((1, _N_CODES), lambda i: (0, 0)),
        ],
        out_specs=pl.BlockSpec((_BT, 1), lambda i: (i, 0)),
        out_shape=jax.ShapeDtypeStruct((n_tok, 1), jnp.int32),
    )(a, xb16, tb16, c, ci)


def _sc_gather(templat, zidx_flat):
    n_tok = zidx_flat.shape[0]
    info = plsc.get_sparse_core_info()
    nw = info.num_cores * info.num_subcores
    b_per_w = n_tok // nw
    chunk = 128
    n_rounds = b_per_w // chunk
    mesh = plsc.VectorSubcoreMesh(core_axis_name="c", subcore_axis_name="s")

    @functools.partial(
        pl.kernel, mesh=mesh,
        out_type=jax.ShapeDtypeStruct((n_tok, _D), jnp.float32),
        scratch_types=[
            pltpu.VMEM((b_per_w,), jnp.int32),
            pltpu.VMEM((chunk, _D), jnp.float32),
            pltpu.VMEM((chunk, _D), jnp.float32),
            pltpu.SemaphoreType.DMA,
            pltpu.SemaphoreType.DMA,
        ],
    )
    def gather(table_hbm, idx_hbm, out_hbm, idx_v, rows_a, rows_b, sem_a,
               sem_b):
        wid = lax.axis_index("s") * info.num_cores + lax.axis_index("c")
        base = wid * b_per_w
        pltpu.sync_copy(idx_hbm.at[pl.ds(base, b_per_w)], idx_v)
        bufs = ((rows_a, sem_a), (rows_b, sem_b))

        def issue(r):
            buf, sem = bufs[r % 2]
            return pltpu.async_copy(
                table_hbm.at[idx_v.at[pl.ds(r * chunk, chunk)]], buf, sem)

        def drain(r, cp):
            cp.wait()
            pltpu.sync_copy(bufs[r % 2][0],
                            out_hbm.at[pl.ds(base + r * chunk, chunk)])

        cps = [issue(0)]
        for r in range(1, n_rounds):
            cps.append(issue(r))
            drain(r - 1, cps[r - 1])
        drain(n_rounds - 1, cps[-1])

    return gather(templat, zidx_flat)


def kernel(input, templat):
    x2d = input.reshape(-1, _D)
    a = jnp.sum(x2d * x2d, axis=-1, keepdims=True)
    c = jnp.sum(templat * templat, axis=-1)[None, :]
    zidx2d = _argmin_codes(x2d, templat, a, c)
    zidx = zidx2d.reshape(input.shape[0], input.shape[1])
    quant2d = _sc_gather(templat, zidx2d.reshape(-1))
    quant = quant2d.reshape(input.shape[0], input.shape[1], _D)
    return (quant, zidx)


# speedup vs baseline: 1.0134x; 1.0134x over previous
import functools

import jax
import jax.numpy as jnp
from jax import lax
from jax.experimental import pallas as pl
from jax.experimental.pallas import tpu as pltpu
from jax.experimental.pallas import tpu_sc as plsc

_N_CODES = 8192
_D = 256
_BT = 1024
_BC = 256


def _dist_argmin_body(a_ref, x_ref, t_ref, c_ref, ci_ref, idx_ref):
    xb = (2.0 * x_ref[...]).astype(jnp.bfloat16)
    a = a_ref[...]
    n_chunks = _N_CODES // _BC

    def dist_at(base):
        tb = t_ref[pl.ds(base, _BC), :]
        prod = lax.dot_general(
            xb, tb, (((1,), (1,)), ((), ())),
            preferred_element_type=jnp.float32,
        )
        return (a - prod) + c_ref[:, pl.ds(base, _BC)]

    def merged_pair(base):
        d0 = dist_at(base)
        d1 = dist_at(base + _BC)
        i0 = jnp.broadcast_to(ci_ref[:, pl.ds(base, _BC)], (_BT, _BC))
        i1 = jnp.broadcast_to(ci_ref[:, pl.ds(base + _BC, _BC)],
                              (_BT, _BC))
        m = d1 < d0
        return jnp.where(m, d1, d0), jnp.where(m, i1, i0)

    def merged_quad(base):
        d0, i0 = merged_pair(base)
        d1, i1 = merged_pair(base + 2 * _BC)
        m = d1 < d0
        return jnp.where(m, d1, d0), jnp.where(m, i1, i0)

    def merged_oct(base):
        d0, i0 = merged_quad(base)
        d1, i1 = merged_quad(base + 4 * _BC)
        m = d1 < d0
        return jnp.where(m, d1, d0), jnp.where(m, i1, i0)

    def merge(a0, a1):
        (d0, i0), (d1, i1) = a0, a1
        m = d1 < d0
        return jnp.where(m, d1, d0), jnp.where(m, i1, i0)

    level = [merged_oct(k * 8 * _BC) for k in range(n_chunks // 8)]
    while len(level) > 1:
        level = [merge(level[k], level[k + 1])
                 for k in range(0, len(level), 2)]
    best, bidx = level[0]
    gmin = jnp.min(best, axis=1, keepdims=True)
    cand = jnp.where(best == gmin, bidx, jnp.float32(_N_CODES))
    idx_ref[...] = jnp.min(cand, axis=1).reshape(_BT, 1).astype(jnp.int32)


def _argmin_codes(x2d, templat, a, c):
    n_tok = x2d.shape[0]
    tb16 = templat.astype(jnp.bfloat16)
    ci = lax.iota(jnp.float32, _N_CODES)[None, :]
    grid = (n_tok // _BT,)
    return pl.pallas_call(
        _dist_argmin_body,
        grid=grid,
        in_specs=[
            pl.BlockSpec((_BT, 1), lambda i: (i, 0)),
            pl.BlockSpec((_BT, _D), lambda i: (i, 0)),
            pl.BlockSpec((_N_CODES, _D), lambda i: (0, 0)),
            pl.BlockSpec((1, _N_CODES), lambda i: (0, 0)),
            pl.BlockSpec((1, _N_CODES), lambda i: (0, 0)),
        ],
        out_specs=pl.BlockSpec((_BT, 1), lambda i: (i, 0)),
        out_shape=jax.ShapeDtypeStruct((n_tok, 1), jnp.int32),
    )(a, x2d, tb16, c, ci)


def _sc_gather(templat, zidx_flat):
    n_tok = zidx_flat.shape[0]
    info = plsc.get_sparse_core_info()
    nw = info.num_cores * info.num_subcores
    b_per_w = n_tok // nw
    chunk = 128
    n_rounds = b_per_w // chunk
    mesh = plsc.VectorSubcoreMesh(core_axis_name="c", subcore_axis_name="s")

    @functools.partial(
        pl.kernel, mesh=mesh,
        out_type=jax.ShapeDtypeStruct((n_tok, _D), jnp.float32),
        scratch_types=[
            pltpu.VMEM((b_per_w,), jnp.int32),
            pltpu.VMEM((chunk, _D), jnp.float32),
            pltpu.VMEM((chunk, _D), jnp.float32),
            pltpu.SemaphoreType.DMA,
            pltpu.SemaphoreType.DMA,
        ],
    )
    def gather(table_hbm, idx_hbm, out_hbm, idx_v, rows_a, rows_b, sem_a,
               sem_b):
        wid = lax.axis_index("s") * info.num_cores + lax.axis_index("c")
        base = wid * b_per_w
        pltpu.sync_copy(idx_hbm.at[pl.ds(base, b_per_w)], idx_v)
        bufs = ((rows_a, sem_a), (rows_b, sem_b))

        def issue(r):
            buf, sem = bufs[r % 2]
            return pltpu.async_copy(
                table_hbm.at[idx_v.at[pl.ds(r * chunk, chunk)]], buf, sem)

        def drain(r, cp):
            cp.wait()
            pltpu.sync_copy(bufs[r % 2][0],
                            out_hbm.at[pl.ds(base + r * chunk, chunk)])

        cps = [issue(0)]
        for r in range(1, n_rounds):
            cps.append(issue(r))
            drain(r - 1, cps[r - 1])
        drain(n_rounds - 1, cps[-1])

    return gather(templat, zidx_flat)


def kernel(input, templat):
    x2d = input.reshape(-1, _D)
    a = jnp.sum(x2d * x2d, axis=-1, keepdims=True)
    c = jnp.sum(templat * templat, axis=-1)[None, :]
    zidx2d = _argmin_codes(x2d, templat, a, c)
    zidx = zidx2d.reshape(input.shape[0], input.shape[1])
    quant2d = _sc_gather(templat, zidx2d.reshape(-1))
    quant = quant2d.reshape(input.shape[0], input.shape[1], _D)
    return (quant, zidx)
